# baseline (speedup 1.0000x reference)
import functools

import jax
import jax.numpy as jnp
from jax import lax
from jax.experimental import pallas as pl
from jax.experimental.pallas import tpu as pltpu
from jax.experimental.pallas import tpu_sc as plsc

N = 10000
D = 128
E = 320000

NC = 2
NSUB = 16
NW = NC * NSUB
EPW = E // NW
CH = 128
CB = 1
NCH = 80
NCA = NCH // CB
NCAH = NCA // 2
EPAD = NCH * CH
PAD = N
ROWS = 10112
RPS = ROWS // NSUB
ZR = 158


def _sc_deg_body(s1, d1, s2, d2, s3, d3, s4, d4, ones_hbm, zeros_hbm,
            out_hbm, sidx, didx, ones, zbuf, vbuf, acc_o, acc_i):
    cid = lax.axis_index("c")
    sid = lax.axis_index("s")
    wid = cid * NSUB + sid
    base = sid * RPS
    pltpu.sync_copy(ones_hbm, ones)
    pltpu.sync_copy(zeros_hbm, zbuf)
    pltpu.sync_copy(zbuf, acc_o.at[pl.ds(base, RPS)])
    pltpu.sync_copy(zbuf, acc_i.at[pl.ds(base, RPS)])
    plsc.subcore_barrier()
    for g, (s_h, d_h) in enumerate(((s1, d1), (s2, d2), (s3, d3), (s4, d4))):
        pltpu.sync_copy(s_h.at[wid], sidx)
        pltpu.sync_copy(d_h.at[wid], didx)

        def chunk(j, carry):
            pltpu.sync_copy(ones, acc_o.at[sidx.at[j]], add=True)
            pltpu.sync_copy(ones, acc_i.at[didx.at[j]], add=True)
            return carry

        lax.fori_loop(0, NCA, chunk, 0)
        plsc.subcore_barrier()
        pltpu.sync_copy(acc_o.at[pl.ds(base, RPS)], vbuf)
        pltpu.sync_copy(vbuf, out_hbm.at[g, cid * 2, pl.ds(base, RPS)])
        pltpu.sync_copy(acc_i.at[pl.ds(base, RPS)], vbuf)
        pltpu.sync_copy(vbuf, out_hbm.at[g, cid * 2 + 1, pl.ds(base, RPS)])
        if g < 3:
            pltpu.sync_copy(zbuf, acc_o.at[pl.ds(base, RPS)])
            pltpu.sync_copy(zbuf, acc_i.at[pl.ds(base, RPS)])
            plsc.subcore_barrier()


def _sc_agg_body(s1, d1, t1, s2, d2, t2, s3, d3, t3, s4, d4, t4, zeros_hbm,
                 out_hbm, sidx, didx, rows, acc, gsem):
    cid = lax.axis_index("c")
    sid = lax.axis_index("s")
    wid = cid * NSUB + sid
    base = sid * RPS
    nfull = RPS // (CB * CH)
    sizes = (CB * CH,) * nfull + ((RPS % (CB * CH),) if RPS % (CB * CH) else ())

    def zero_acc():
        pltpu.sync_copy(zeros_hbm, rows)
        o = 0
        for sz in sizes:
            pltpu.sync_copy(rows.at[pl.ds(0, sz)],
                            acc.at[pl.ds(base + o, sz)])
            o += sz

    zero_acc()
    plsc.subcore_barrier()
    for g, (s_h, d_h, t_h) in enumerate(
        ((s1, d1, t1), (s2, d2, t2), (s3, d3, t3), (s4, d4, t4))
    ):
        for h in range(2):
            pltpu.sync_copy(s_h.at[wid, pl.ds(h * NCAH, NCAH)], sidx)
            pltpu.sync_copy(d_h.at[wid, pl.ds(h * NCAH, NCAH)], didx)

            def chunk(j, carry):
                pltpu.async_copy(t_h.at[sidx.at[j]], rows, gsem).wait()
                pltpu.sync_copy(rows, acc.at[didx.at[j]], add=True)
                return carry

            lax.fori_loop(0, NCAH, chunk, 0)
        plsc.subcore_barrier()
        o = 0
        for sz in sizes:
            pltpu.sync_copy(acc.at[pl.ds(base + o, sz)],
                            rows.at[pl.ds(0, sz)])
            pltpu.sync_copy(rows.at[pl.ds(0, sz)],
                            out_hbm.at[g, cid, pl.ds(base + o, sz)])
            o += sz
        if g < 3:
            zero_acc()
            plsc.subcore_barrier()


@functools.lru_cache(maxsize=1)
def _sc_kernels():
    mesh = plsc.VectorSubcoreMesh(
        core_axis_name="c", subcore_axis_name="s",
        num_cores=NC, num_subcores=NSUB,
    )
    params = pltpu.CompilerParams(use_tc_tiling_on_sc=False)
    sc_deg = pl.kernel(
        _sc_deg_body,
        out_type=jax.ShapeDtypeStruct((4, 4, ROWS, 16), jnp.float32),
        mesh=mesh,
        compiler_params=params,
        scratch_types=[
            pltpu.VMEM((NCA, CB * CH), jnp.int32),
            pltpu.VMEM((NCA, CB * CH), jnp.int32),
            pltpu.VMEM((CB * CH, 16), jnp.float32),
            pltpu.VMEM((RPS, 16), jnp.float32),
            pltpu.VMEM((RPS, 16), jnp.float32),
            pltpu.VMEM_SHARED((ROWS, 16), jnp.float32),
            pltpu.VMEM_SHARED((ROWS, 16), jnp.float32),
        ],
    )
    sc_agg = pl.kernel(
        _sc_agg_body,
        out_type=jax.ShapeDtypeStruct((4, 2, ROWS, D), jnp.float32),
        mesh=mesh,
        compiler_params=params,
        scratch_types=[
            pltpu.VMEM((NCAH, CB * CH), jnp.int32),
            pltpu.VMEM((NCAH, CB * CH), jnp.int32),
            pltpu.VMEM((CB * CH, D), jnp.float32),
            pltpu.VMEM_SHARED((ROWS, D), jnp.float32),
            pltpu.SemaphoreType.DMA,
        ],
    )
    return sc_deg, sc_agg


def _prep_body(x_ref, degc_ref, xs_ref, r2_ref):
    deg = degc_ref[0]
    r_out = lax.rsqrt(jnp.maximum(deg[0] + deg[2], 1.0))
    r_in = lax.rsqrt(jnp.maximum(deg[1] + deg[3], 1.0))
    r2_ref[0, 0] = r_out
    r2_ref[0, 1] = r_in
    xs_ref[0] = x_ref[0] * r_out[:, None]


_prep = pl.pallas_call(
    _prep_body,
    grid=(4,),
    in_specs=[
        pl.BlockSpec((1, ROWS, D), lambda g: (g, 0, 0)),
        pl.BlockSpec((1, 4, ROWS), lambda g: (g, 0, 0)),
    ],
    out_specs=[
        pl.BlockSpec((1, ROWS, D), lambda g: (g, 0, 0)),
        pl.BlockSpec((1, 2, ROWS), lambda g: (g, 0, 0)),
    ],
    out_shape=[
        jax.ShapeDtypeStruct((4, ROWS, D), jnp.float32),
        jax.ShapeDtypeStruct((4, 2, ROWS), jnp.float32),
    ],
)


def _layer_body(agg_ref, r2_ref, w_ref, b_ref, ys_ref):
    a = agg_ref[0, 0] + agg_ref[0, 1]
    a = a * r2_ref[0, 1][:, None]
    z = jnp.dot(a, w_ref[...], preferred_element_type=jnp.float32)
    z = jnp.maximum(z + b_ref[0], 0.0)
    ys_ref[0] = z * r2_ref[0, 0][:, None]


_layer = pl.pallas_call(
    _layer_body,
    grid=(4,),
    in_specs=[
        pl.BlockSpec((1, 2, ROWS, D), lambda g: (g, 0, 0, 0)),
        pl.BlockSpec((1, 2, ROWS), lambda g: (g, 0, 0)),
        pl.BlockSpec((D, D), lambda g: (0, 0)),
        pl.BlockSpec((1, D), lambda g: (0, 0)),
    ],
    out_specs=pl.BlockSpec((1, ROWS, D), lambda g: (g, 0, 0)),
    out_shape=jax.ShapeDtypeStruct((4, ROWS, D), jnp.float32),
)


def _final_body(agg_ref, r2_ref, w_ref, b_ref, out_ref):
    a = agg_ref[0, 0] + agg_ref[0, 1]
    a = a * r2_ref[0, 1][:, None]
    z = jnp.dot(a, w_ref[...], preferred_element_type=jnp.float32)
    z = jnp.maximum(z + b_ref[0], 0.0)

    @pl.when(pl.program_id(0) == 0)
    def _():
        out_ref[...] = jnp.zeros_like(out_ref)

    out_ref[0, :] += jnp.sum(z[:N, :], axis=0)


_final = pl.pallas_call(
    _final_body,
    grid=(4,),
    in_specs=[
        pl.BlockSpec((1, 2, ROWS, D), lambda g: (g, 0, 0, 0)),
        pl.BlockSpec((1, 2, ROWS), lambda g: (g, 0, 0)),
        pl.BlockSpec((D, D), lambda g: (0, 0)),
        pl.BlockSpec((1, D), lambda g: (0, 0)),
    ],
    out_specs=pl.BlockSpec((1, D), lambda g: (0, 0)),
    out_shape=jax.ShapeDtypeStruct((1, D), jnp.float32),
)


def _pad_edges(row):
    r = row.reshape(NW, EPW)
    r = jnp.pad(r, ((0, 0), (0, EPAD - EPW)), constant_values=PAD)
    return r.reshape(NW, NCA, CB * CH)


def kernel(g1, x1, g2, x2, g3, x3, g4, x4, W1, b1, W2, b2):
    srcs = [_pad_edges(g[0]) for g in (g1, g2, g3, g4)]
    dsts = [_pad_edges(g[1]) for g in (g1, g2, g3, g4)]
    x = jnp.stack([x1, x2, x3, x4])
    x = jnp.pad(x, ((0, 0), (0, ROWS - N), (0, 0)))

    ones16 = jnp.ones((CB * CH, 16), jnp.float32)
    zeros16 = jnp.zeros((RPS, 16), jnp.float32)
    zerosD = jnp.zeros((CB * CH, D), jnp.float32)
    b1r = b1.reshape(1, D)
    b2r = b2.reshape(1, D)

    sc_deg, sc_agg = _sc_kernels()
    deg = sc_deg(srcs[0], dsts[0], srcs[1], dsts[1], srcs[2], dsts[2],
                 srcs[3], dsts[3], ones16, zeros16)
    degc = deg[:, :, :, 0]

    xs, r2 = _prep(x, degc)
    a1 = sc_agg(srcs[0], dsts[0], xs[0], srcs[1], dsts[1], xs[1],
                srcs[2], dsts[2], xs[2], srcs[3], dsts[3], xs[3], zerosD)
    ys = _layer(a1, r2, W1, b1r)
    a2 = sc_agg(srcs[0], dsts[0], ys[0], srcs[1], dsts[1], ys[1],
                srcs[2], dsts[2], ys[2], srcs[3], dsts[3], ys[3], zerosD)
    tot = _final(a2, r2, W2, b2r)
    return jnp.sum(tot) * (1.0 / (4.0 * N * D))


# speedup vs baseline: 1.0022x; 1.0022x over previous
import functools

import jax
import jax.numpy as jnp
from jax import lax
from jax.experimental import pallas as pl
from jax.experimental.pallas import tpu as pltpu
from jax.experimental.pallas import tpu_sc as plsc

N = 10000
D = 128
E = 320000

NC = 2
NSUB = 16
NW = NC * NSUB
EPW = E // NW
CH = 128
CB = 1
NCH = 80
NCA = NCH // CB
NCAH = NCA
EPAD = NCH * CH
PAD = N
ROWS = 10112
RPS = ROWS // NSUB
ZR = 158


def _sc_deg_body(s1, d1, s2, d2, s3, d3, s4, d4, ones_hbm, zeros_hbm,
            out_hbm, sidx, didx, ones, zbuf, vbuf, acc_o, acc_i):
    cid = lax.axis_index("c")
    sid = lax.axis_index("s")
    wid = cid * NSUB + sid
    base = sid * RPS
    pltpu.sync_copy(ones_hbm, ones)
    pltpu.sync_copy(zeros_hbm, zbuf)
    pltpu.sync_copy(zbuf, acc_o.at[pl.ds(base, RPS)])
    pltpu.sync_copy(zbuf, acc_i.at[pl.ds(base, RPS)])
    plsc.subcore_barrier()
    for g, (s_h, d_h) in enumerate(((s1, d1), (s2, d2), (s3, d3), (s4, d4))):
        pltpu.sync_copy(s_h.at[wid], sidx)
        pltpu.sync_copy(d_h.at[wid], didx)

        def chunk(j, carry):
            pltpu.sync_copy(ones, acc_o.at[sidx.at[j]], add=True)
            pltpu.sync_copy(ones, acc_i.at[didx.at[j]], add=True)
            return carry

        lax.fori_loop(0, NCA, chunk, 0)
        plsc.subcore_barrier()
        pltpu.sync_copy(acc_o.at[pl.ds(base, RPS)], vbuf)
        pltpu.sync_copy(vbuf, out_hbm.at[g, cid * 2, pl.ds(base, RPS)])
        pltpu.sync_copy(acc_i.at[pl.ds(base, RPS)], vbuf)
        pltpu.sync_copy(vbuf, out_hbm.at[g, cid * 2 + 1, pl.ds(base, RPS)])
        if g < 3:
            pltpu.sync_copy(zbuf, acc_o.at[pl.ds(base, RPS)])
            pltpu.sync_copy(zbuf, acc_i.at[pl.ds(base, RPS)])
            plsc.subcore_barrier()


def _sc_agg_body(s1, d1, t1, s2, d2, t2, s3, d3, t3, s4, d4, t4, zeros_hbm,
                 out_hbm, sidx, didx, rows, acc, gsem):
    cid = lax.axis_index("c")
    sid = lax.axis_index("s")
    wid = cid * NSUB + sid
    base = sid * RPS
    nfull = RPS // (CB * CH)
    sizes = (CB * CH,) * nfull + ((RPS % (CB * CH),) if RPS % (CB * CH) else ())

    def zero_acc():
        pltpu.sync_copy(zeros_hbm, rows)
        o = 0
        for sz in sizes:
            pltpu.sync_copy(rows.at[pl.ds(0, sz)],
                            acc.at[pl.ds(base + o, sz)])
            o += sz

    zero_acc()
    plsc.subcore_barrier()
    for g, (s_h, d_h, t_h) in enumerate(
        ((s1, d1, t1), (s2, d2, t2), (s3, d3, t3), (s4, d4, t4))
    ):
        for h in range(1):
            pltpu.sync_copy(s_h.at[wid, pl.ds(h * NCAH, NCAH)], sidx)
            pltpu.sync_copy(d_h.at[wid, pl.ds(h * NCAH, NCAH)], didx)

            def chunk(j, carry):
                pltpu.async_copy(t_h.at[sidx.at[j]], rows, gsem).wait()
                pltpu.sync_copy(rows, acc.at[didx.at[j]], add=True)
                return carry

            lax.fori_loop(0, NCAH, chunk, 0)
        plsc.subcore_barrier()
        o = 0
        for sz in sizes:
            pltpu.sync_copy(acc.at[pl.ds(base + o, sz)],
                            rows.at[pl.ds(0, sz)])
            pltpu.sync_copy(rows.at[pl.ds(0, sz)],
                            out_hbm.at[g, cid, pl.ds(base + o, sz)])
            o += sz
        if g < 3:
            zero_acc()
            plsc.subcore_barrier()


@functools.lru_cache(maxsize=1)
def _sc_kernels():
    mesh = plsc.VectorSubcoreMesh(
        core_axis_name="c", subcore_axis_name="s",
        num_cores=NC, num_subcores=NSUB,
    )
    params = pltpu.CompilerParams(use_tc_tiling_on_sc=False)
    sc_deg = pl.kernel(
        _sc_deg_body,
        out_type=jax.ShapeDtypeStruct((4, 4, ROWS, 16), jnp.float32),
        mesh=mesh,
        compiler_params=params,
        scratch_types=[
            pltpu.VMEM((NCA, CB * CH), jnp.int32),
            pltpu.VMEM((NCA, CB * CH), jnp.int32),
            pltpu.VMEM((CB * CH, 16), jnp.float32),
            pltpu.VMEM((RPS, 16), jnp.float32),
            pltpu.VMEM((RPS, 16), jnp.float32),
            pltpu.VMEM_SHARED((ROWS, 16), jnp.float32),
            pltpu.VMEM_SHARED((ROWS, 16), jnp.float32),
        ],
    )
    sc_agg = pl.kernel(
        _sc_agg_body,
        out_type=jax.ShapeDtypeStruct((4, 2, ROWS, D), jnp.float32),
        mesh=mesh,
        compiler_params=params,
        scratch_types=[
            pltpu.VMEM((NCAH, CB * CH), jnp.int32),
            pltpu.VMEM((NCAH, CB * CH), jnp.int32),
            pltpu.VMEM((CB * CH, D), jnp.float32),
            pltpu.VMEM_SHARED((ROWS, D), jnp.float32),
            pltpu.SemaphoreType.DMA,
        ],
    )
    return sc_deg, sc_agg


def _prep_body(x_ref, degc_ref, xs_ref, r2_ref):
    deg = degc_ref[0]
    r_out = lax.rsqrt(jnp.maximum(deg[0] + deg[2], 1.0))
    r_in = lax.rsqrt(jnp.maximum(deg[1] + deg[3], 1.0))
    r2_ref[0, 0] = r_out
    r2_ref[0, 1] = r_in
    xs_ref[0] = x_ref[0] * r_out[:, None]


_prep = pl.pallas_call(
    _prep_body,
    grid=(4,),
    in_specs=[
        pl.BlockSpec((1, ROWS, D), lambda g: (g, 0, 0)),
        pl.BlockSpec((1, 4, ROWS), lambda g: (g, 0, 0)),
    ],
    out_specs=[
        pl.BlockSpec((1, ROWS, D), lambda g: (g, 0, 0)),
        pl.BlockSpec((1, 2, ROWS), lambda g: (g, 0, 0)),
    ],
    out_shape=[
        jax.ShapeDtypeStruct((4, ROWS, D), jnp.float32),
        jax.ShapeDtypeStruct((4, 2, ROWS), jnp.float32),
    ],
)


def _layer_body(agg_ref, r2_ref, w_ref, b_ref, ys_ref):
    a = agg_ref[0, 0] + agg_ref[0, 1]
    a = a * r2_ref[0, 1][:, None]
    z = jnp.dot(a, w_ref[...], preferred_element_type=jnp.float32)
    z = jnp.maximum(z + b_ref[0], 0.0)
    ys_ref[0] = z * r2_ref[0, 0][:, None]


_layer = pl.pallas_call(
    _layer_body,
    grid=(4,),
    in_specs=[
        pl.BlockSpec((1, 2, ROWS, D), lambda g: (g, 0, 0, 0)),
        pl.BlockSpec((1, 2, ROWS), lambda g: (g, 0, 0)),
        pl.BlockSpec((D, D), lambda g: (0, 0)),
        pl.BlockSpec((1, D), lambda g: (0, 0)),
    ],
    out_specs=pl.BlockSpec((1, ROWS, D), lambda g: (g, 0, 0)),
    out_shape=jax.ShapeDtypeStruct((4, ROWS, D), jnp.float32),
)


def _final_body(agg_ref, r2_ref, w_ref, b_ref, out_ref):
    a = agg_ref[0, 0] + agg_ref[0, 1]
    a = a * r2_ref[0, 1][:, None]
    z = jnp.dot(a, w_ref[...], preferred_element_type=jnp.float32)
    z = jnp.maximum(z + b_ref[0], 0.0)

    @pl.when(pl.program_id(0) == 0)
    def _():
        out_ref[...] = jnp.zeros_like(out_ref)

    out_ref[0, :] += jnp.sum(z[:N, :], axis=0)


_final = pl.pallas_call(
    _final_body,
    grid=(4,),
    in_specs=[
        pl.BlockSpec((1, 2, ROWS, D), lambda g: (g, 0, 0, 0)),
        pl.BlockSpec((1, 2, ROWS), lambda g: (g, 0, 0)),
        pl.BlockSpec((D, D), lambda g: (0, 0)),
        pl.BlockSpec((1, D), lambda g: (0, 0)),
    ],
    out_specs=pl.BlockSpec((1, D), lambda g: (0, 0)),
    out_shape=jax.ShapeDtypeStruct((1, D), jnp.float32),
)


def _pad_edges(row):
    r = row.reshape(NW, EPW)
    r = jnp.pad(r, ((0, 0), (0, EPAD - EPW)), constant_values=PAD)
    return r.reshape(NW, NCA, CB * CH)


def kernel(g1, x1, g2, x2, g3, x3, g4, x4, W1, b1, W2, b2):
    srcs = [_pad_edges(g[0]) for g in (g1, g2, g3, g4)]
    dsts = [_pad_edges(g[1]) for g in (g1, g2, g3, g4)]
    x = jnp.stack([x1, x2, x3, x4])
    x = jnp.pad(x, ((0, 0), (0, ROWS - N), (0, 0)))

    ones16 = jnp.ones((CB * CH, 16), jnp.float32)
    zeros16 = jnp.zeros((RPS, 16), jnp.float32)
    zerosD = jnp.zeros((CB * CH, D), jnp.float32)
    b1r = b1.reshape(1, D)
    b2r = b2.reshape(1, D)

    sc_deg, sc_agg = _sc_kernels()
    deg = sc_deg(srcs[0], dsts[0], srcs[1], dsts[1], srcs[2], dsts[2],
                 srcs[3], dsts[3], ones16, zeros16)
    degc = deg[:, :, :, 0]

    xs, r2 = _prep(x, degc)
    a1 = sc_agg(srcs[0], dsts[0], xs[0], srcs[1], dsts[1], xs[1],
                srcs[2], dsts[2], xs[2], srcs[3], dsts[3], xs[3], zerosD)
    ys = _layer(a1, r2, W1, b1r)
    a2 = sc_agg(srcs[0], dsts[0], ys[0], srcs[1], dsts[1], ys[1],
                srcs[2], dsts[2], ys[2], srcs[3], dsts[3], ys[3], zerosD)
    tot = _final(a2, r2, W2, b2r)
    return jnp.sum(tot) * (1.0 / (4.0 * N * D))


# speedup vs baseline: 2.2618x; 2.2569x over previous
import functools

import jax
import jax.numpy as jnp
from jax import lax
from jax.experimental import pallas as pl
from jax.experimental.pallas import tpu as pltpu
from jax.experimental.pallas import tpu_sc as plsc

N = 10000
D = 128
E = 320000

NC = 2
NSUB = 16
NW = NC * NSUB
EPW = E // NW
CH = 128
CB = 1
NCH = 79
NCA = NCH // CB
NCAH = NCA
EPAD = NCH * CH
PAD = N
ROWS = 10112
RPS = ROWS // NSUB
ZR = 158


def _sc_deg_body(s1, d1, s2, d2, s3, d3, s4, d4, ones_hbm, zeros_hbm,
            out_hbm, sidx, didx, ones, zbuf, vbuf, acc_o, acc_i):
    cid = lax.axis_index("c")
    sid = lax.axis_index("s")
    wid = cid * NSUB + sid
    base = sid * RPS
    pltpu.sync_copy(ones_hbm, ones)
    pltpu.sync_copy(zeros_hbm, zbuf)
    pltpu.sync_copy(zbuf, acc_o.at[pl.ds(base, RPS)])
    pltpu.sync_copy(zbuf, acc_i.at[pl.ds(base, RPS)])
    plsc.subcore_barrier()
    for g, (s_h, d_h) in enumerate(((s1, d1), (s2, d2), (s3, d3), (s4, d4))):
        pltpu.sync_copy(s_h.at[wid], sidx)
        pltpu.sync_copy(d_h.at[wid], didx)

        def chunk(j, carry):
            pltpu.sync_copy(ones, acc_o.at[sidx.at[j]], add=True)
            pltpu.sync_copy(ones, acc_i.at[didx.at[j]], add=True)
            return carry

        lax.fori_loop(0, NCA, chunk, 0)
        plsc.subcore_barrier()
        pltpu.sync_copy(acc_o.at[pl.ds(base, RPS)], vbuf)
        pltpu.sync_copy(vbuf, out_hbm.at[g, cid * 2, pl.ds(base, RPS)])
        pltpu.sync_copy(acc_i.at[pl.ds(base, RPS)], vbuf)
        pltpu.sync_copy(vbuf, out_hbm.at[g, cid * 2 + 1, pl.ds(base, RPS)])
        if g < 3:
            pltpu.sync_copy(zbuf, acc_o.at[pl.ds(base, RPS)])
            pltpu.sync_copy(zbuf, acc_i.at[pl.ds(base, RPS)])
            plsc.subcore_barrier()


def _sc_agg_body(s1, d1, t1, s2, d2, t2, s3, d3, t3, s4, d4, t4, zeros_hbm,
                 out_hbm, sidx, didx, rows, acc, gsem):
    cid = lax.axis_index("c")
    sid = lax.axis_index("s")
    wid = cid * NSUB + sid
    base = sid * RPS
    nfull = RPS // (CB * CH)
    sizes = (CB * CH,) * nfull + ((RPS % (CB * CH),) if RPS % (CB * CH) else ())

    def zero_acc():
        pltpu.sync_copy(zeros_hbm, rows)
        o = 0
        for sz in sizes:
            pltpu.sync_copy(rows.at[pl.ds(0, sz)],
                            acc.at[pl.ds(base + o, sz)])
            o += sz

    zero_acc()
    plsc.subcore_barrier()
    for g, (s_h, d_h, t_h) in enumerate(
        ((s1, d1, t1), (s2, d2, t2), (s3, d3, t3), (s4, d4, t4))
    ):
        for h in range(1):
            pltpu.sync_copy(s_h.at[wid, pl.ds(h * NCAH, NCAH)], sidx)
            pltpu.sync_copy(d_h.at[wid, pl.ds(h * NCAH, NCAH)], didx)

            def chunk(j, carry):
                pltpu.async_copy(t_h.at[sidx.at[j]], rows, gsem).wait()
                pltpu.sync_copy(rows, acc.at[didx.at[j]], add=True)
                return carry

            lax.fori_loop(0, NCAH, chunk, 0)
        plsc.subcore_barrier()
        o = 0
        for sz in sizes:
            pltpu.sync_copy(acc.at[pl.ds(base + o, sz)],
                            rows.at[pl.ds(0, sz)])
            pltpu.sync_copy(rows.at[pl.ds(0, sz)],
                            out_hbm.at[g, cid, pl.ds(base + o, sz)])
            o += sz
        if g < 3:
            zero_acc()
            plsc.subcore_barrier()


@functools.lru_cache(maxsize=1)
def _sc_kernels():
    mesh = plsc.VectorSubcoreMesh(
        core_axis_name="c", subcore_axis_name="s",
        num_cores=NC, num_subcores=NSUB,
    )
    params = pltpu.CompilerParams(use_tc_tiling_on_sc=False)
    sc_deg = pl.kernel(
        _sc_deg_body,
        out_type=jax.ShapeDtypeStruct((4, 4, ROWS, 16), jnp.float32),
        mesh=mesh,
        compiler_params=params,
        scratch_types=[
            pltpu.VMEM((NCA, CB * CH), jnp.int32),
            pltpu.VMEM((NCA, CB * CH), jnp.int32),
            pltpu.VMEM((CB * CH, 16), jnp.float32),
            pltpu.VMEM((RPS, 16), jnp.float32),
            pltpu.VMEM((RPS, 16), jnp.float32),
            pltpu.VMEM_SHARED((ROWS, 16), jnp.float32),
            pltpu.VMEM_SHARED((ROWS, 16), jnp.float32),
        ],
    )
    sc_agg = pl.kernel(
        _sc_agg_body,
        out_type=jax.ShapeDtypeStruct((4, 2, ROWS, D), jnp.float32),
        mesh=mesh,
        compiler_params=params,
        scratch_types=[
            pltpu.VMEM((NCAH, CB * CH), jnp.int32),
            pltpu.VMEM((NCAH, CB * CH), jnp.int32),
            pltpu.VMEM((CB * CH, D), jnp.float32),
            pltpu.VMEM_SHARED((ROWS, D), jnp.float32),
            pltpu.SemaphoreType.DMA,
        ],
    )
    return sc_deg, sc_agg


def _prep_body(x_ref, degc_ref, xs_ref, r2_ref):
    deg = degc_ref[0]
    r_out = lax.rsqrt(jnp.maximum(deg[0] + deg[2], 1.0))
    r_in = lax.rsqrt(jnp.maximum(deg[1] + deg[3], 1.0))
    r2_ref[0, 0] = r_out
    r2_ref[0, 1] = r_in
    xs_ref[0] = x_ref[0] * r_out[:, None]


_prep = pl.pallas_call(
    _prep_body,
    grid=(4,),
    in_specs=[
        pl.BlockSpec((1, ROWS, D), lambda g: (g, 0, 0)),
        pl.BlockSpec((1, 4, ROWS), lambda g: (g, 0, 0)),
    ],
    out_specs=[
        pl.BlockSpec((1, ROWS, D), lambda g: (g, 0, 0)),
        pl.BlockSpec((1, 2, ROWS), lambda g: (g, 0, 0)),
    ],
    out_shape=[
        jax.ShapeDtypeStruct((4, ROWS, D), jnp.float32),
        jax.ShapeDtypeStruct((4, 2, ROWS), jnp.float32),
    ],
)


def _layer_body(agg_ref, r2_ref, w_ref, b_ref, ys_ref):
    a = agg_ref[0, 0] + agg_ref[0, 1]
    a = a * r2_ref[0, 1][:, None]
    z = jnp.dot(a, w_ref[...], preferred_element_type=jnp.float32)
    z = jnp.maximum(z + b_ref[0], 0.0)
    ys_ref[0] = z * r2_ref[0, 0][:, None]


_layer = pl.pallas_call(
    _layer_body,
    grid=(4,),
    in_specs=[
        pl.BlockSpec((1, 2, ROWS, D), lambda g: (g, 0, 0, 0)),
        pl.BlockSpec((1, 2, ROWS), lambda g: (g, 0, 0)),
        pl.BlockSpec((D, D), lambda g: (0, 0)),
        pl.BlockSpec((1, D), lambda g: (0, 0)),
    ],
    out_specs=pl.BlockSpec((1, ROWS, D), lambda g: (g, 0, 0)),
    out_shape=jax.ShapeDtypeStruct((4, ROWS, D), jnp.float32),
)


def _final_body(agg_ref, r2_ref, w_ref, b_ref, out_ref):
    a = agg_ref[0, 0] + agg_ref[0, 1]
    a = a * r2_ref[0, 1][:, None]
    z = jnp.dot(a, w_ref[...], preferred_element_type=jnp.float32)
    z = jnp.maximum(z + b_ref[0], 0.0)

    @pl.when(pl.program_id(0) == 0)
    def _():
        out_ref[...] = jnp.zeros_like(out_ref)

    out_ref[0, :] += jnp.sum(z[:N, :], axis=0)


_final = pl.pallas_call(
    _final_body,
    grid=(4,),
    in_specs=[
        pl.BlockSpec((1, 2, ROWS, D), lambda g: (g, 0, 0, 0)),
        pl.BlockSpec((1, 2, ROWS), lambda g: (g, 0, 0)),
        pl.BlockSpec((D, D), lambda g: (0, 0)),
        pl.BlockSpec((1, D), lambda g: (0, 0)),
    ],
    out_specs=pl.BlockSpec((1, D), lambda g: (0, 0)),
    out_shape=jax.ShapeDtypeStruct((1, D), jnp.float32),
)


def _pad_edges(row):
    r = row.reshape(NW, EPW)
    pad = N + (jnp.arange(EPAD - EPW, dtype=jnp.int32) % (ROWS - N))
    pad = jnp.broadcast_to(pad, (NW, EPAD - EPW))
    return jnp.concatenate([r, pad], axis=1).reshape(NW, NCA, CB * CH)


def kernel(g1, x1, g2, x2, g3, x3, g4, x4, W1, b1, W2, b2):
    srcs = [_pad_edges(g[0]) for g in (g1, g2, g3, g4)]
    dsts = [_pad_edges(g[1]) for g in (g1, g2, g3, g4)]
    x = jnp.stack([x1, x2, x3, x4])
    x = jnp.pad(x, ((0, 0), (0, ROWS - N), (0, 0)))

    ones16 = jnp.ones((CB * CH, 16), jnp.float32)
    zeros16 = jnp.zeros((RPS, 16), jnp.float32)
    zerosD = jnp.zeros((CB * CH, D), jnp.float32)
    b1r = b1.reshape(1, D)
    b2r = b2.reshape(1, D)

    sc_deg, sc_agg = _sc_kernels()
    deg = sc_deg(srcs[0], dsts[0], srcs[1], dsts[1], srcs[2], dsts[2],
                 srcs[3], dsts[3], ones16, zeros16)
    degc = deg[:, :, :, 0]

    xs, r2 = _prep(x, degc)
    a1 = sc_agg(srcs[0], dsts[0], xs[0], srcs[1], dsts[1], xs[1],
                srcs[2], dsts[2], xs[2], srcs[3], dsts[3], xs[3], zerosD)
    ys = _layer(a1, r2, W1, b1r)
    a2 = sc_agg(srcs[0], dsts[0], ys[0], srcs[1], dsts[1], ys[1],
                srcs[2], dsts[2], ys[2], srcs[3], dsts[3], ys[3], zerosD)
    tot = _final(a2, r2, W2, b2r)
    return jnp.sum(tot) * (1.0 / (4.0 * N * D))


# speedup vs baseline: 3.1825x; 1.4071x over previous
import functools

import jax
import jax.numpy as jnp
from jax import lax
from jax.experimental import pallas as pl
from jax.experimental.pallas import tpu as pltpu
from jax.experimental.pallas import tpu_sc as plsc

N = 10000
D = 128
E = 320000

NC = 2
NSUB = 16
NW = NC * NSUB
EPW = E // NW
CH = 120
NCA = 84
BLK = 12
NBL = NCA // BLK
EPAD = NCA * CH
PAD = N
ROWS = 10112
RPS = ROWS // NSUB
ZR = 158


def _sc_deg_body(s1, d1, s2, d2, s3, d3, s4, d4, ones_hbm, zeros_hbm,
            out_hbm, sidx, didx, ones, zbuf, vbuf, acc_o, acc_i,
            dsem_a, dsem_b):
    cid = lax.axis_index("c")
    sid = lax.axis_index("s")
    wid = cid * NSUB + sid
    base = sid * RPS
    pltpu.sync_copy(ones_hbm, ones)
    pltpu.sync_copy(zeros_hbm, zbuf)
    pltpu.sync_copy(zbuf, acc_o.at[pl.ds(base, RPS)])
    pltpu.sync_copy(zbuf, acc_i.at[pl.ds(base, RPS)])
    plsc.subcore_barrier()
    for g, (s_h, d_h) in enumerate(((s1, d1), (s2, d2), (s3, d3), (s4, d4))):
        pltpu.sync_copy(s_h.at[wid], sidx)
        pltpu.sync_copy(d_h.at[wid], didx)

        def chunk(j, carry):
            d1 = pltpu.async_copy(ones, acc_o.at[sidx.at[j]], dsem_a, add=True)
            d2 = pltpu.async_copy(ones, acc_i.at[didx.at[j]], dsem_b, add=True)
            d1.wait()
            d2.wait()
            return carry

        lax.fori_loop(0, NCA, chunk, 0)
        plsc.subcore_barrier()
        pltpu.sync_copy(acc_o.at[pl.ds(base, RPS)], vbuf)
        pltpu.sync_copy(vbuf, out_hbm.at[g, cid * 2, pl.ds(base, RPS)])
        pltpu.sync_copy(acc_i.at[pl.ds(base, RPS)], vbuf)
        pltpu.sync_copy(vbuf, out_hbm.at[g, cid * 2 + 1, pl.ds(base, RPS)])
        if g < 3:
            pltpu.sync_copy(zbuf, acc_o.at[pl.ds(base, RPS)])
            pltpu.sync_copy(zbuf, acc_i.at[pl.ds(base, RPS)])
            plsc.subcore_barrier()


def _sc_agg_body(s1, d1, s2, d2, s3, d3, s4, d4, tabs, zeros_hbm,
                 out_hbm, sidx, didx, rows, rows_b, rows_c, acc,
                 sem_a, sem_b, sem_c):
    cid = lax.axis_index("c")
    sid = lax.axis_index("s")
    wid = cid * NSUB + sid
    base = sid * RPS
    nfull = RPS // CH
    sizes = (CH,) * nfull + ((RPS % CH,) if RPS % CH else ())

    def zero_acc():
        pltpu.sync_copy(zeros_hbm, rows)
        o = 0
        for sz in sizes:
            pltpu.sync_copy(rows.at[pl.ds(0, sz)],
                            acc.at[pl.ds(base + o, sz)])
            o += sz

    zero_acc()
    plsc.subcore_barrier()
    for g, (s_h, d_h) in enumerate(((s1, d1), (s2, d2), (s3, d3), (s4, d4))):
        t_h = tabs.at[g]
        bufs = ((rows, sem_a), (rows_b, sem_b), (rows_c, sem_c))

        def gather(j, buf, sem):
            return pltpu.async_copy(t_h.at[sidx.at[j]], buf, sem)

        def scatter(j, buf):
            pltpu.sync_copy(buf, acc.at[didx.at[j]], add=True)

        def block(b, carry):
            pltpu.sync_copy(s_h.at[wid, pl.ds(b * BLK, BLK)], sidx)
            pltpu.sync_copy(d_h.at[wid, pl.ds(b * BLK, BLK)], didx)
            for o, (buf, sem) in enumerate(bufs):
                gather(o, buf, sem)

            def triple(t, c2):
                j = 3 * t
                for o, (buf, sem) in enumerate(bufs):
                    pltpu.make_async_copy(t_h.at[sidx.at[j + o]], buf,
                                          sem).wait()
                    scatter(j + o, buf)
                    gather(j + 3 + o, buf, sem)
                return c2

            lax.fori_loop(0, BLK // 3 - 1, triple, 0)
            j = BLK - 3
            for o, (buf, sem) in enumerate(bufs):
                pltpu.make_async_copy(t_h.at[sidx.at[j + o]], buf, sem).wait()
                scatter(j + o, buf)
            return carry

        lax.fori_loop(0, NBL, block, 0)
        plsc.subcore_barrier()
        o = 0
        for sz in sizes:
            pltpu.sync_copy(acc.at[pl.ds(base + o, sz)],
                            rows.at[pl.ds(0, sz)])
            pltpu.sync_copy(rows.at[pl.ds(0, sz)],
                            out_hbm.at[g, cid, pl.ds(base + o, sz)])
            o += sz
        if g < 3:
            zero_acc()
            plsc.subcore_barrier()


@functools.lru_cache(maxsize=1)
def _sc_kernels():
    mesh = plsc.VectorSubcoreMesh(
        core_axis_name="c", subcore_axis_name="s",
        num_cores=NC, num_subcores=NSUB,
    )
    params = pltpu.CompilerParams(use_tc_tiling_on_sc=False)
    sc_deg = pl.kernel(
        _sc_deg_body,
        out_type=jax.ShapeDtypeStruct((4, 4, ROWS, 16), jnp.float32),
        mesh=mesh,
        compiler_params=params,
        scratch_types=[
            pltpu.VMEM((NCA, CH), jnp.int32),
            pltpu.VMEM((NCA, CH), jnp.int32),
            pltpu.VMEM((CH, 16), jnp.float32),
            pltpu.VMEM((RPS, 16), jnp.float32),
            pltpu.VMEM((RPS, 16), jnp.float32),
            pltpu.VMEM_SHARED((ROWS, 16), jnp.float32),
            pltpu.VMEM_SHARED((ROWS, 16), jnp.float32),
            pltpu.SemaphoreType.DMA,
            pltpu.SemaphoreType.DMA,
        ],
    )
    sc_agg = pl.kernel(
        _sc_agg_body,
        out_type=jax.ShapeDtypeStruct((4, 2, ROWS, D), jnp.float32),
        mesh=mesh,
        compiler_params=params,
        scratch_types=[
            pltpu.VMEM((BLK, CH), jnp.int32),
            pltpu.VMEM((BLK, CH), jnp.int32),
            pltpu.VMEM((CH, D), jnp.float32),
            pltpu.VMEM((CH, D), jnp.float32),
            pltpu.VMEM((CH, D), jnp.float32),
            pltpu.VMEM_SHARED((ROWS, D), jnp.float32),
            pltpu.SemaphoreType.DMA,
            pltpu.SemaphoreType.DMA,
            pltpu.SemaphoreType.DMA,
        ],
    )
    return sc_deg, sc_agg


def _prep_body(x_ref, deg_ref, xs_ref, r2_ref):
    deg = deg_ref[0]
    r_out = lax.rsqrt(jnp.maximum(deg[0] + deg[2], 1.0))
    r_in = lax.rsqrt(jnp.maximum(deg[1] + deg[3], 1.0))
    r2_ref[0, 0] = r_out
    r2_ref[0, 1] = r_in
    xs_ref[0, :N] = x_ref[0] * r_out[:N, None]


_prep = pl.pallas_call(
    _prep_body,
    grid=(4,),
    in_specs=[
        pl.BlockSpec((1, N, D), lambda g: (g, 0, 0)),
        pl.BlockSpec((1, 4, ROWS), lambda g: (g, 0, 0)),
    ],
    out_specs=[
        pl.BlockSpec((1, ROWS, D), lambda g: (g, 0, 0)),
        pl.BlockSpec((1, 2, ROWS), lambda g: (g, 0, 0)),
    ],
    out_shape=[
        jax.ShapeDtypeStruct((4, ROWS, D), jnp.float32),
        jax.ShapeDtypeStruct((4, 2, ROWS), jnp.float32),
    ],
)


def _layer_body(agg_ref, r2_ref, w_ref, b_ref, ys_ref):
    a = agg_ref[0, 0] + agg_ref[0, 1]
    a = a * r2_ref[0, 1][:, None]
    z = jnp.dot(a, w_ref[...], preferred_element_type=jnp.float32)
    z = jnp.maximum(z + b_ref[0], 0.0)
    ys_ref[0] = z * r2_ref[0, 0][:, None]


_layer = pl.pallas_call(
    _layer_body,
    grid=(4,),
    in_specs=[
        pl.BlockSpec((1, 2, ROWS, D), lambda g: (g, 0, 0, 0)),
        pl.BlockSpec((1, 2, ROWS), lambda g: (g, 0, 0)),
        pl.BlockSpec((D, D), lambda g: (0, 0)),
        pl.BlockSpec((1, D), lambda g: (0, 0)),
    ],
    out_specs=pl.BlockSpec((1, ROWS, D), lambda g: (g, 0, 0)),
    out_shape=jax.ShapeDtypeStruct((4, ROWS, D), jnp.float32),
)


def _final_body(agg_ref, r2_ref, w_ref, b_ref, out_ref):
    a = agg_ref[0, 0] + agg_ref[0, 1]
    a = a * r2_ref[0, 1][:, None]
    z = jnp.dot(a, w_ref[...], preferred_element_type=jnp.float32)
    z = jnp.maximum(z + b_ref[0], 0.0)

    @pl.when(pl.program_id(0) == 0)
    def _():
        out_ref[...] = jnp.zeros_like(out_ref)

    out_ref[0, :] += jnp.sum(z[:N, :], axis=0)


_final = pl.pallas_call(
    _final_body,
    grid=(4,),
    in_specs=[
        pl.BlockSpec((1, 2, ROWS, D), lambda g: (g, 0, 0, 0)),
        pl.BlockSpec((1, 2, ROWS), lambda g: (g, 0, 0)),
        pl.BlockSpec((D, D), lambda g: (0, 0)),
        pl.BlockSpec((1, D), lambda g: (0, 0)),
    ],
    out_specs=pl.BlockSpec((1, D), lambda g: (0, 0)),
    out_shape=jax.ShapeDtypeStruct((1, D), jnp.float32),
)


def _pad_edges(row):
    r = row.reshape(NW, EPW)
    pad = N + (jnp.arange(EPAD - EPW, dtype=jnp.int32) % (ROWS - N))
    pad = jnp.broadcast_to(pad, (NW, EPAD - EPW))
    return jnp.concatenate([r, pad], axis=1).reshape(NW, NCA, CH)


def kernel(g1, x1, g2, x2, g3, x3, g4, x4, W1, b1, W2, b2):
    srcs = [_pad_edges(g[0]) for g in (g1, g2, g3, g4)]
    dsts = [_pad_edges(g[1]) for g in (g1, g2, g3, g4)]
    x = jnp.stack([x1, x2, x3, x4])

    ones16 = jnp.ones((CH, 16), jnp.float32)
    zeros16 = jnp.zeros((RPS, 16), jnp.float32)
    zerosD = jnp.zeros((CH, D), jnp.float32)
    b1r = b1.reshape(1, D)
    b2r = b2.reshape(1, D)

    sc_deg, sc_agg = _sc_kernels()
    deg = sc_deg(srcs[0], dsts[0], srcs[1], dsts[1], srcs[2], dsts[2],
                 srcs[3], dsts[3], ones16, zeros16)
    degc = deg[:, :, :, 0]

    xs, r2 = _prep(x, degc)
    a1 = sc_agg(srcs[0], dsts[0], srcs[1], dsts[1],
                srcs[2], dsts[2], srcs[3], dsts[3], xs, zerosD)
    ys = _layer(a1, r2, W1, b1r)
    a2 = sc_agg(srcs[0], dsts[0], srcs[1], dsts[1],
                srcs[2], dsts[2], srcs[3], dsts[3], ys, zerosD)
    tot = _final(a2, r2, W2, b2r)
    return jnp.sum(tot) * (1.0 / (4.0 * N * D))


# speedup vs baseline: 3.3257x; 1.0450x over previous
import functools

import jax
import jax.numpy as jnp
from jax import lax
from jax.experimental import pallas as pl
from jax.experimental.pallas import tpu as pltpu
from jax.experimental.pallas import tpu_sc as plsc

N = 10000
D = 128
E = 320000

NC = 2
NSUB = 16
NW = NC * NSUB
EPW = E // NW
CH = 120
NCA = 84
BLK = 12
NBL = NCA // BLK
EPAD = NCA * CH
EPW2 = E // NSUB
NCA2 = 168
NBL2 = NCA2 // BLK
EPAD2 = NCA2 * CH
PAD = N
ROWS = 10112
RPS = ROWS // NSUB
ZR = 158


def _sc_deg_body(s1, d1, s2, d2, s3, d3, s4, d4, ones_hbm, zeros_hbm,
            out_hbm, sidx, didx, ones, zbuf, vbuf, acc_o, acc_i,
            dsem_a, dsem_b):
    cid = lax.axis_index("c")
    sid = lax.axis_index("s")
    wid = cid * NSUB + sid
    base = sid * RPS
    pltpu.sync_copy(ones_hbm, ones)
    pltpu.sync_copy(zeros_hbm, zbuf)
    pltpu.sync_copy(zbuf, acc_o.at[pl.ds(base, RPS)])
    pltpu.sync_copy(zbuf, acc_i.at[pl.ds(base, RPS)])
    plsc.subcore_barrier()
    for g, (s_h, d_h) in enumerate(((s1, d1), (s2, d2), (s3, d3), (s4, d4))):
        pltpu.sync_copy(s_h.at[wid], sidx)
        pltpu.sync_copy(d_h.at[wid], didx)

        def chunk(j, carry):
            d1 = pltpu.async_copy(ones, acc_o.at[sidx.at[j]], dsem_a, add=True)
            d2 = pltpu.async_copy(ones, acc_i.at[didx.at[j]], dsem_b, add=True)
            d1.wait()
            d2.wait()
            return carry

        lax.fori_loop(0, NCA, chunk, 0)
        plsc.subcore_barrier()
        pltpu.sync_copy(acc_o.at[pl.ds(base, RPS)], vbuf)
        pltpu.sync_copy(vbuf, out_hbm.at[g, cid * 2, pl.ds(base, RPS)])
        pltpu.sync_copy(acc_i.at[pl.ds(base, RPS)], vbuf)
        pltpu.sync_copy(vbuf, out_hbm.at[g, cid * 2 + 1, pl.ds(base, RPS)])
        if g < 3:
            pltpu.sync_copy(zbuf, acc_o.at[pl.ds(base, RPS)])
            pltpu.sync_copy(zbuf, acc_i.at[pl.ds(base, RPS)])
            plsc.subcore_barrier()


def _sc_agg_body(s1, d1, s2, d2, s3, d3, s4, d4, tabs, zeros_hbm,
                 out_hbm, sidx, didx, rows, rows_b, rows_c, acc,
                 sem_a, sem_b, sem_c):
    cid = lax.axis_index("c")
    sid = lax.axis_index("s")
    base = sid * RPS
    nfull = RPS // CH
    sizes = (CH,) * nfull + ((RPS % CH,) if RPS % CH else ())

    def zero_acc():
        pltpu.sync_copy(zeros_hbm, rows)
        o = 0
        for sz in sizes:
            pltpu.sync_copy(rows.at[pl.ds(0, sz)],
                            acc.at[pl.ds(base + o, sz)])
            o += sz

    zero_acc()
    plsc.subcore_barrier()
    edges = ((s1, d1), (s2, d2), (s3, d3), (s4, d4))
    bufs = ((rows, sem_a), (rows_b, sem_b), (rows_c, sem_c))
    for r in range(2):
        for c in range(NC):
            g = 2 * c + r
            s_h, d_h = edges[g]
            t_h = tabs.at[g]

            def gather(j, buf, sem, t_h=t_h):
                return pltpu.async_copy(t_h.at[sidx.at[j]], buf, sem)

            def scatter(j, buf):
                pltpu.sync_copy(buf, acc.at[didx.at[j]], add=True)

            @pl.when(cid == c)
            def _process(s_h=s_h, d_h=d_h, t_h=t_h, gather=gather,
                         scatter=scatter):
                def block(b, carry):
                    pltpu.sync_copy(s_h.at[sid, pl.ds(b * BLK, BLK)], sidx)
                    pltpu.sync_copy(d_h.at[sid, pl.ds(b * BLK, BLK)], didx)
                    for o, (buf, sem) in enumerate(bufs):
                        gather(o, buf, sem)

                    def triple(t, c2):
                        j = 3 * t
                        for o, (buf, sem) in enumerate(bufs):
                            pltpu.make_async_copy(t_h.at[sidx.at[j + o]], buf,
                                                  sem).wait()
                            scatter(j + o, buf)
                            gather(j + 3 + o, buf, sem)
                        return c2

                    lax.fori_loop(0, BLK // 3 - 1, triple, 0)
                    j = BLK - 3
                    for o, (buf, sem) in enumerate(bufs):
                        pltpu.make_async_copy(t_h.at[sidx.at[j + o]], buf,
                                              sem).wait()
                        scatter(j + o, buf)
                    return carry

                lax.fori_loop(0, NBL2, block, 0)

        plsc.subcore_barrier()
        for c in range(NC):
            g = 2 * c + r

            @pl.when(cid == c)
            def _copy_out(g=g):
                o = 0
                for sz in sizes:
                    pltpu.sync_copy(acc.at[pl.ds(base + o, sz)],
                                    rows.at[pl.ds(0, sz)])
                    pltpu.sync_copy(rows.at[pl.ds(0, sz)],
                                    out_hbm.at[g, pl.ds(base + o, sz)])
                    o += sz
        if r == 0:
            zero_acc()
            plsc.subcore_barrier()


@functools.lru_cache(maxsize=1)
def _sc_kernels():
    mesh = plsc.VectorSubcoreMesh(
        core_axis_name="c", subcore_axis_name="s",
        num_cores=NC, num_subcores=NSUB,
    )
    params = pltpu.CompilerParams(use_tc_tiling_on_sc=False)
    sc_deg = pl.kernel(
        _sc_deg_body,
        out_type=jax.ShapeDtypeStruct((4, 4, ROWS, 16), jnp.float32),
        mesh=mesh,
        compiler_params=params,
        scratch_types=[
            pltpu.VMEM((NCA, CH), jnp.int32),
            pltpu.VMEM((NCA, CH), jnp.int32),
            pltpu.VMEM((CH, 16), jnp.float32),
            pltpu.VMEM((RPS, 16), jnp.float32),
            pltpu.VMEM((RPS, 16), jnp.float32),
            pltpu.VMEM_SHARED((ROWS, 16), jnp.float32),
            pltpu.VMEM_SHARED((ROWS, 16), jnp.float32),
            pltpu.SemaphoreType.DMA,
            pltpu.SemaphoreType.DMA,
        ],
    )
    sc_agg = pl.kernel(
        _sc_agg_body,
        out_type=jax.ShapeDtypeStruct((4, ROWS, D), jnp.float32),
        mesh=mesh,
        compiler_params=params,
        scratch_types=[
            pltpu.VMEM((BLK, CH), jnp.int32),
            pltpu.VMEM((BLK, CH), jnp.int32),
            pltpu.VMEM((CH, D), jnp.float32),
            pltpu.VMEM((CH, D), jnp.float32),
            pltpu.VMEM((CH, D), jnp.float32),
            pltpu.VMEM_SHARED((ROWS, D), jnp.float32),
            pltpu.SemaphoreType.DMA,
            pltpu.SemaphoreType.DMA,
            pltpu.SemaphoreType.DMA,
        ],
    )
    return sc_deg, sc_agg


def _prep_body(x_ref, deg_ref, xs_ref, r2_ref):
    deg = deg_ref[0]
    r_out = lax.rsqrt(jnp.maximum(deg[0] + deg[2], 1.0))
    r_in = lax.rsqrt(jnp.maximum(deg[1] + deg[3], 1.0))
    r2_ref[0, 0] = r_out
    r2_ref[0, 1] = r_in
    xs_ref[0, :N] = x_ref[0] * r_out[:N, None]


_prep = pl.pallas_call(
    _prep_body,
    grid=(4,),
    in_specs=[
        pl.BlockSpec((1, N, D), lambda g: (g, 0, 0)),
        pl.BlockSpec((1, 4, ROWS), lambda g: (g, 0, 0)),
    ],
    out_specs=[
        pl.BlockSpec((1, ROWS, D), lambda g: (g, 0, 0)),
        pl.BlockSpec((1, 2, ROWS), lambda g: (g, 0, 0)),
    ],
    out_shape=[
        jax.ShapeDtypeStruct((4, ROWS, D), jnp.float32),
        jax.ShapeDtypeStruct((4, 2, ROWS), jnp.float32),
    ],
)


def _layer_body(agg_ref, r2_ref, w_ref, b_ref, ys_ref):
    a = agg_ref[0]
    a = a * r2_ref[0, 1][:, None]
    z = jnp.dot(a, w_ref[...], preferred_element_type=jnp.float32)
    z = jnp.maximum(z + b_ref[0], 0.0)
    ys_ref[0] = z * r2_ref[0, 0][:, None]


_layer = pl.pallas_call(
    _layer_body,
    grid=(4,),
    in_specs=[
        pl.BlockSpec((1, ROWS, D), lambda g: (g, 0, 0)),
        pl.BlockSpec((1, 2, ROWS), lambda g: (g, 0, 0)),
        pl.BlockSpec((D, D), lambda g: (0, 0)),
        pl.BlockSpec((1, D), lambda g: (0, 0)),
    ],
    out_specs=pl.BlockSpec((1, ROWS, D), lambda g: (g, 0, 0)),
    out_shape=jax.ShapeDtypeStruct((4, ROWS, D), jnp.float32),
)


def _final_body(agg_ref, r2_ref, w_ref, b_ref, out_ref):
    a = agg_ref[0]
    a = a * r2_ref[0, 1][:, None]
    z = jnp.dot(a, w_ref[...], preferred_element_type=jnp.float32)
    z = jnp.maximum(z + b_ref[0], 0.0)

    @pl.when(pl.program_id(0) == 0)
    def _():
        out_ref[...] = jnp.zeros_like(out_ref)

    out_ref[0, :] += jnp.sum(z[:N, :], axis=0)


_final = pl.pallas_call(
    _final_body,
    grid=(4,),
    in_specs=[
        pl.BlockSpec((1, ROWS, D), lambda g: (g, 0, 0)),
        pl.BlockSpec((1, 2, ROWS), lambda g: (g, 0, 0)),
        pl.BlockSpec((D, D), lambda g: (0, 0)),
        pl.BlockSpec((1, D), lambda g: (0, 0)),
    ],
    out_specs=pl.BlockSpec((1, D), lambda g: (0, 0)),
    out_shape=jax.ShapeDtypeStruct((1, D), jnp.float32),
)


def _pad_edges(row):
    r = row.reshape(NW, EPW)
    pad = N + (jnp.arange(EPAD - EPW, dtype=jnp.int32) % (ROWS - N))
    pad = jnp.broadcast_to(pad, (NW, EPAD - EPW))
    return jnp.concatenate([r, pad], axis=1).reshape(NW, NCA, CH)


def _pad_edges2(row):
    r = row.reshape(NSUB, EPW2)
    pad = N + (jnp.arange(EPAD2 - EPW2, dtype=jnp.int32) % (ROWS - N))
    pad = jnp.broadcast_to(pad, (NSUB, EPAD2 - EPW2))
    return jnp.concatenate([r, pad], axis=1).reshape(NSUB, NCA2, CH)


def kernel(g1, x1, g2, x2, g3, x3, g4, x4, W1, b1, W2, b2):
    srcs = [_pad_edges(g[0]) for g in (g1, g2, g3, g4)]
    dsts = [_pad_edges(g[1]) for g in (g1, g2, g3, g4)]
    srcs2 = [_pad_edges2(g[0]) for g in (g1, g2, g3, g4)]
    dsts2 = [_pad_edges2(g[1]) for g in (g1, g2, g3, g4)]
    x = jnp.stack([x1, x2, x3, x4])

    ones16 = jnp.ones((CH, 16), jnp.float32)
    zeros16 = jnp.zeros((RPS, 16), jnp.float32)
    zerosD = jnp.zeros((CH, D), jnp.float32)
    b1r = b1.reshape(1, D)
    b2r = b2.reshape(1, D)

    sc_deg, sc_agg = _sc_kernels()
    deg = sc_deg(srcs[0], dsts[0], srcs[1], dsts[1], srcs[2], dsts[2],
                 srcs[3], dsts[3], ones16, zeros16)
    degc = deg[:, :, :, 0]

    xs, r2 = _prep(x, degc)
    a1 = sc_agg(srcs2[0], dsts2[0], srcs2[1], dsts2[1],
                srcs2[2], dsts2[2], srcs2[3], dsts2[3], xs, zerosD)
    ys = _layer(a1, r2, W1, b1r)
    a2 = sc_agg(srcs2[0], dsts2[0], srcs2[1], dsts2[1],
                srcs2[2], dsts2[2], srcs2[3], dsts2[3], ys, zerosD)
    tot = _final(a2, r2, W2, b2r)
    return jnp.sum(tot) * (1.0 / (4.0 * N * D))
